# baseline (speedup 1.0000x reference)
import functools

import jax
import jax.numpy as jnp
from jax import lax
from jax.experimental import pallas as pl
from jax.experimental.pallas import tpu as pltpu
from jax.experimental.pallas import tpu_sc as plsc

NUM_TOKENS = 16384
D_MODEL = 2048
NUM_EXPERTS = 64
TOP_K = 2


BT = 2048


def _probs_kernel(x_ref, w_ref, b_ref, probs_ref):
    logits = (
        jnp.dot(x_ref[...], w_ref[...], preferred_element_type=jnp.float32)
        + b_ref[...]
    )
    m = jnp.max(logits, axis=-1, keepdims=True)
    e = jnp.exp(logits - m)
    probs_ref[...] = e / jnp.sum(e, axis=-1, keepdims=True)


def _tc_probs(inputs, W, b2):
    return pl.pallas_call(
        _probs_kernel,
        grid=(NUM_TOKENS // BT,),
        in_specs=[
            pl.BlockSpec((BT, D_MODEL), lambda i: (i, 0)),
            pl.BlockSpec((D_MODEL, NUM_EXPERTS), lambda i: (0, 0)),
            pl.BlockSpec((1, NUM_EXPERTS), lambda i: (0, 0)),
        ],
        out_specs=pl.BlockSpec((BT, NUM_EXPERTS), lambda i: (i, 0)),
        out_shape=jax.ShapeDtypeStruct((NUM_TOKENS, NUM_EXPERTS), jnp.float32),
    )(inputs, W, b2)


NC = 2
NS = 16
NW = NC * NS
TOK_PER_W = NUM_TOKENS // NW
GROUPS = TOK_PER_W // 16

_sc_mesh = plsc.VectorSubcoreMesh(core_axis_name="c", subcore_axis_name="s")


@functools.partial(
    pl.kernel,
    out_type=[
        jax.ShapeDtypeStruct((NUM_TOKENS,), jnp.float32),
        jax.ShapeDtypeStruct((NUM_TOKENS,), jnp.float32),
        jax.ShapeDtypeStruct((NUM_TOKENS,), jnp.int32),
        jax.ShapeDtypeStruct((NUM_TOKENS,), jnp.int32),
    ],
    mesh=_sc_mesh,
    compiler_params=pltpu.CompilerParams(needs_layout_passes=False),
    scratch_types=[
        pltpu.VMEM((TOK_PER_W * NUM_EXPERTS,), jnp.float32),
        pltpu.VMEM((TOK_PER_W,), jnp.float32),
        pltpu.VMEM((TOK_PER_W,), jnp.float32),
        pltpu.VMEM((TOK_PER_W,), jnp.int32),
        pltpu.VMEM((TOK_PER_W,), jnp.int32),
    ],
)
def _sc_top2(probs_hbm, p1_hbm, p2_hbm, i1_hbm, i2_hbm,
             probs_v, p1_v, p2_v, i1_v, i2_v):
    wid = lax.axis_index("s") * NC + lax.axis_index("c")
    base = wid * TOK_PER_W
    pltpu.sync_copy(
        probs_hbm.at[pl.ds(base * NUM_EXPERTS, TOK_PER_W * NUM_EXPERTS)],
        probs_v,
    )

    lane = lax.iota(jnp.int32, 16)

    def group_body(g, carry):
        tok = g * 16 + lane
        m1 = jnp.full((16,), -1.0, jnp.float32)
        m2 = jnp.full((16,), -1.0, jnp.float32)
        i1 = jnp.zeros((16,), jnp.int32)
        i2 = jnp.zeros((16,), jnp.int32)
        row = tok * NUM_EXPERTS
        for e in range(NUM_EXPERTS):
            v = plsc.load_gather(probs_v, [row + e])
            gt1 = v > m1
            gt2 = v > m2
            ecur = jnp.full((16,), e, jnp.int32)
            m2 = jnp.where(gt1, m1, jnp.where(gt2, v, m2))
            i2 = jnp.where(gt1, i1, jnp.where(gt2, ecur, i2))
            m1 = jnp.where(gt1, v, m1)
            i1 = jnp.where(gt1, ecur, i1)
        s = m1 + m2
        sl = pl.ds(g * 16, 16)
        p1_v[sl] = m1 / s
        p2_v[sl] = m2 / s
        i1_v[sl] = i1
        i2_v[sl] = i2
        return carry

    lax.fori_loop(0, GROUPS, group_body, 0)

    out_sl = pl.ds(base, TOK_PER_W)
    pltpu.sync_copy(p1_v, p1_hbm.at[out_sl])
    pltpu.sync_copy(p2_v, p2_hbm.at[out_sl])
    pltpu.sync_copy(i1_v, i1_hbm.at[out_sl])
    pltpu.sync_copy(i2_v, i2_hbm.at[out_sl])


@jax.jit
def kernel(inputs, W, b):
    b2 = b.reshape(1, NUM_EXPERTS)
    probs = _tc_probs(inputs, W, b2)
    p1, p2, i1, i2 = _sc_top2(probs.reshape(-1))
    topk = jnp.stack([p1, p2], axis=-1)
    idx = jnp.stack([i1, i2], axis=-1)
    return (topk, idx, probs)


# speedup vs baseline: 1.2055x; 1.2055x over previous
import functools

import jax
import jax.numpy as jnp
from jax import lax
from jax.experimental import pallas as pl
from jax.experimental.pallas import tpu as pltpu
from jax.experimental.pallas import tpu_sc as plsc

NUM_TOKENS = 16384
D_MODEL = 2048
NUM_EXPERTS = 64
TOP_K = 2


BT = 2048


def _probs_kernel(x_ref, w_ref, b_ref, probs_ref, probs_t_ref):
    logits = (
        jnp.dot(x_ref[...], w_ref[...], preferred_element_type=jnp.float32)
        + b_ref[...]
    )
    m = jnp.max(logits, axis=-1, keepdims=True)
    e = jnp.exp(logits - m)
    probs = e / jnp.sum(e, axis=-1, keepdims=True)
    probs_ref[...] = probs
    probs_t_ref[...] = probs.T


def _tc_probs(inputs, W, b2):
    return pl.pallas_call(
        _probs_kernel,
        grid=(NUM_TOKENS // BT,),
        in_specs=[
            pl.BlockSpec((BT, D_MODEL), lambda i: (i, 0)),
            pl.BlockSpec((D_MODEL, NUM_EXPERTS), lambda i: (0, 0)),
            pl.BlockSpec((1, NUM_EXPERTS), lambda i: (0, 0)),
        ],
        out_specs=[
            pl.BlockSpec((BT, NUM_EXPERTS), lambda i: (i, 0)),
            pl.BlockSpec((NUM_EXPERTS, BT), lambda i: (0, i)),
        ],
        out_shape=[
            jax.ShapeDtypeStruct((NUM_TOKENS, NUM_EXPERTS), jnp.float32),
            jax.ShapeDtypeStruct((NUM_EXPERTS, NUM_TOKENS), jnp.float32),
        ],
    )(inputs, W, b2)


NC = 2
NS = 16
NW = NC * NS
TOK_PER_W = NUM_TOKENS // NW
GROUPS = TOK_PER_W // 16

_sc_mesh = plsc.VectorSubcoreMesh(core_axis_name="c", subcore_axis_name="s")


@functools.partial(
    pl.kernel,
    out_type=[
        jax.ShapeDtypeStruct((NUM_TOKENS,), jnp.float32),
        jax.ShapeDtypeStruct((NUM_TOKENS,), jnp.float32),
        jax.ShapeDtypeStruct((NUM_TOKENS,), jnp.int32),
        jax.ShapeDtypeStruct((NUM_TOKENS,), jnp.int32),
    ],
    mesh=_sc_mesh,
    compiler_params=pltpu.CompilerParams(needs_layout_passes=False),
    scratch_types=[
        pltpu.VMEM((NUM_EXPERTS, TOK_PER_W), jnp.float32),
        pltpu.VMEM((TOK_PER_W,), jnp.float32),
        pltpu.VMEM((TOK_PER_W,), jnp.float32),
        pltpu.VMEM((TOK_PER_W,), jnp.int32),
        pltpu.VMEM((TOK_PER_W,), jnp.int32),
    ],
)
def _sc_top2(probs_t_hbm, p1_hbm, p2_hbm, i1_hbm, i2_hbm,
             probs_v, p1_v, p2_v, i1_v, i2_v):
    wid = lax.axis_index("s") * NC + lax.axis_index("c")
    base = wid * TOK_PER_W
    pltpu.sync_copy(probs_t_hbm.at[:, pl.ds(base, TOK_PER_W)], probs_v)

    def group_body(g, carry):
        sl = pl.ds(g * 16, 16)
        m1 = jnp.full((16,), -1.0, jnp.float32)
        m2 = jnp.full((16,), -1.0, jnp.float32)
        i1 = jnp.zeros((16,), jnp.int32)
        i2 = jnp.zeros((16,), jnp.int32)
        for e in range(NUM_EXPERTS):
            v = probs_v[e, sl]
            gt1 = v > m1
            gt2 = v > m2
            ecur = jnp.full((16,), e, jnp.int32)
            m2 = jnp.where(gt1, m1, jnp.where(gt2, v, m2))
            i2 = jnp.where(gt1, i1, jnp.where(gt2, ecur, i2))
            m1 = jnp.where(gt1, v, m1)
            i1 = jnp.where(gt1, ecur, i1)
        s = m1 + m2
        p1_v[sl] = m1 / s
        p2_v[sl] = m2 / s
        i1_v[sl] = i1
        i2_v[sl] = i2
        return carry

    lax.fori_loop(0, GROUPS, group_body, 0)

    out_sl = pl.ds(base, TOK_PER_W)
    pltpu.sync_copy(p1_v, p1_hbm.at[out_sl])
    pltpu.sync_copy(p2_v, p2_hbm.at[out_sl])
    pltpu.sync_copy(i1_v, i1_hbm.at[out_sl])
    pltpu.sync_copy(i2_v, i2_hbm.at[out_sl])


@jax.jit
def kernel(inputs, W, b):
    b2 = b.reshape(1, NUM_EXPERTS)
    probs, probs_t = _tc_probs(inputs, W, b2)
    p1, p2, i1, i2 = _sc_top2(probs_t)
    topk = jnp.stack([p1, p2], axis=-1)
    idx = jnp.stack([i1, i2], axis=-1)
    return (topk, idx, probs)
